# Initial kernel scaffold; baseline (speedup 1.0000x reference)
import jax
import jax.numpy as jnp
from jax.experimental import pallas as pl


def kernel(input, adj, weight, bias):
    raise NotImplementedError("write your pallas kernel here")


# speedup vs baseline: 1.0056x; 1.0056x over previous
import jax
import jax.numpy as jnp
from jax.experimental import pallas as pl


def _gcn_body(adj_ref, x_ref, w_ref, b_ref, out_ref):
    support = jnp.dot(adj_ref[...], x_ref[...],
                      preferred_element_type=jnp.float32)
    out_ref[...] = jnp.dot(support, w_ref[...],
                           preferred_element_type=jnp.float32) + b_ref[...]


def kernel(input, adj, weight, bias):
    n_rows, f_in = input.shape
    f_out = weight.shape[1]
    n_dst = adj.shape[0]
    bm = 400

    out = pl.pallas_call(
        _gcn_body,
        grid=(n_dst // bm,),
        in_specs=[
            pl.BlockSpec((bm, n_rows), lambda i: (i, 0)),
            pl.BlockSpec((n_rows, f_in), lambda i: (0, 0)),
            pl.BlockSpec((f_in, f_out), lambda i: (0, 0)),
            pl.BlockSpec((1, f_out), lambda i: (0, 0)),
        ],
        out_specs=pl.BlockSpec((bm, f_out), lambda i: (i, 0)),
        out_shape=jax.ShapeDtypeStruct((n_dst, f_out), jnp.float32),
    )(adj, input, weight, bias.reshape(1, f_out))
    return out
